# speedup vs baseline: 1.2254x; 1.2254x over previous
import jax
import jax.numpy as jnp
from jax import lax
from jax.experimental import pallas as pl
from jax.experimental.pallas import tpu as pltpu


def _body(wr1, br1, wr2, br2, xr, yr, we1, be1, we2, be2, we3, be3, we4, be4,
          ox, oy):
    x = xr[0]
    y = yr[0]
    inv = 1.0 / x.shape[1]

    def _means(v):
        return [jnp.sum(v[c:c + 1, :]) * inv for c in range(v.shape[0])]

    mx = _means(x)
    my = _means(y)

    def _gates(m):
        h0 = wr1[0, 0] * m[0] + wr1[0, 1] * m[1] + wr1[0, 2] * m[2] \
            + wr1[0, 3] * m[3] + br1[0]
        h1 = wr1[1, 0] * m[0] + wr1[1, 1] * m[1] + wr1[1, 2] * m[2] \
            + wr1[1, 3] * m[3] + br1[1]
        g0 = wr2[0, 0] * h0 + wr2[0, 1] * h1 + br2[0]
        g1 = wr2[1, 0] * h0 + wr2[1, 1] * h1 + br2[1]
        return g0, g1

    gx0, gx1 = _gates(mx)
    gy0, gy1 = _gates(my)
    one = jnp.float32(1.0)
    zero = jnp.float32(0.0)
    mx0 = jnp.where(gx0 > 0, one, zero)
    mx1 = jnp.where(gx1 > 0, one, zero)
    my0 = jnp.where(gy0 > 0, one, zero)
    my1 = jnp.where(gy1 > 0, one, zero)

    dn = (((1,), (0,)), ((), ()))

    wx0 = mx0 * we1[...]
    wy0 = my0 * we3[...]
    b0 = mx0 * be1[...] + my0 * be3[...]
    ox[0] = (lax.dot_general(wx0, x, dn, preferred_element_type=jnp.float32)
             + lax.dot_general(wy0, y, dn, preferred_element_type=jnp.float32)
             + b0)

    wx1 = mx1 * we2[...]
    wy1 = my1 * we4[...]
    b1 = mx1 * be2[...] + my1 * be4[...]
    oy[0] = (lax.dot_general(wx1, x, dn, preferred_element_type=jnp.float32)
             + lax.dot_general(wy1, y, dn, preferred_element_type=jnp.float32)
             + b1)


def kernel(x, y, W_r1, b_r1, W_r2, b_r2, W_e1, b_e1, W_e2, b_e2,
           W_e3, b_e3, W_e4, b_e4):
    B, C, H, W = x.shape
    P = H * W
    x2 = x.reshape(B, C, P)
    y2 = y.reshape(B, C, P)
    co_x = W_e1.shape[0]
    co_y = W_e2.shape[0]

    smem = pl.BlockSpec(memory_space=pltpu.SMEM)

    def _vmem_small(shape):
        return pl.BlockSpec(shape, lambda b: (0,) * len(shape))

    def big(c):
        return pl.BlockSpec((1, c, P), lambda b: (b, 0, 0))

    out_x, out_y = pl.pallas_call(
        _body,
        grid=(B,),
        in_specs=[
            smem,
            smem,
            smem,
            smem,
            big(C),
            big(C),
            _vmem_small((co_x, C)),
            _vmem_small((co_x, 1)),
            _vmem_small((co_y, C)),
            _vmem_small((co_y, 1)),
            _vmem_small((co_x, C)),
            _vmem_small((co_x, 1)),
            _vmem_small((co_y, C)),
            _vmem_small((co_y, 1)),
        ],
        out_specs=[big(co_x), big(co_y)],
        out_shape=[
            jax.ShapeDtypeStruct((B, co_x, P), jnp.float32),
            jax.ShapeDtypeStruct((B, co_y, P), jnp.float32),
        ],
        compiler_params=pltpu.CompilerParams(
            dimension_semantics=("arbitrary",),
        ),
    )(W_r1, b_r1, W_r2, b_r2, x2, y2,
      W_e1, b_e1.reshape(co_x, 1), W_e2, b_e2.reshape(co_y, 1),
      W_e3, b_e3.reshape(co_x, 1), W_e4, b_e4.reshape(co_y, 1))

    return out_x.reshape(B, co_x, H, W), out_y.reshape(B, co_y, H, W)


# speedup vs baseline: 3.4192x; 2.7902x over previous
import jax
import jax.numpy as jnp
from jax.experimental import pallas as pl
from jax.experimental.pallas import tpu as pltpu


def _body(wr1, br1, wr2, br2, we1, be1, we2, be2, we3, be3, we4, be4,
          xr, yr, ox, oy):
    x = [xr[0, c] for c in range(xr.shape[1])]
    y = [yr[0, c] for c in range(yr.shape[1])]
    inv = 1.0 / (x[0].shape[0] * x[0].shape[1])

    mx = [jnp.sum(v) * inv for v in x]
    my = [jnp.sum(v) * inv for v in y]

    def _gates(m):
        h0 = wr1[0, 0] * m[0] + wr1[0, 1] * m[1] + wr1[0, 2] * m[2] \
            + wr1[0, 3] * m[3] + br1[0]
        h1 = wr1[1, 0] * m[0] + wr1[1, 1] * m[1] + wr1[1, 2] * m[2] \
            + wr1[1, 3] * m[3] + br1[1]
        g0 = wr2[0, 0] * h0 + wr2[0, 1] * h1 + br2[0]
        g1 = wr2[1, 0] * h0 + wr2[1, 1] * h1 + br2[1]
        return g0, g1

    gx0, gx1 = _gates(mx)
    gy0, gy1 = _gates(my)
    one = jnp.float32(1.0)
    zero = jnp.float32(0.0)
    mx0 = jnp.where(gx0 > 0, one, zero)
    mx1 = jnp.where(gx1 > 0, one, zero)
    my0 = jnp.where(gy0 > 0, one, zero)
    my1 = jnp.where(gy1 > 0, one, zero)

    def _mix(o, wx, bx, wy, by, mskx, msky):
        acc = (mskx * wx[o, 0]) * x[0]
        for c in range(1, len(x)):
            acc = acc + (mskx * wx[o, c]) * x[c]
        for c in range(len(y)):
            acc = acc + (msky * wy[o, c]) * y[c]
        return acc + (mskx * bx[o] + msky * by[o])

    for o in range(ox.shape[1]):
        ox[0, o] = _mix(o, we1, be1, we3, be3, mx0, my0)
    for o in range(oy.shape[1]):
        oy[0, o] = _mix(o, we2, be2, we4, be4, mx1, my1)


def kernel(x, y, W_r1, b_r1, W_r2, b_r2, W_e1, b_e1, W_e2, b_e2,
           W_e3, b_e3, W_e4, b_e4):
    B, C, H, W = x.shape
    co_x = W_e1.shape[0]
    co_y = W_e2.shape[0]

    smem = pl.BlockSpec(memory_space=pltpu.SMEM)

    def big(c):
        return pl.BlockSpec((1, c, H, W), lambda b: (b, 0, 0, 0))

    out_x, out_y = pl.pallas_call(
        _body,
        grid=(B,),
        in_specs=[smem] * 12 + [big(C), big(C)],
        out_specs=[big(co_x), big(co_y)],
        out_shape=[
            jax.ShapeDtypeStruct((B, co_x, H, W), jnp.float32),
            jax.ShapeDtypeStruct((B, co_y, H, W), jnp.float32),
        ],
        compiler_params=pltpu.CompilerParams(
            dimension_semantics=("arbitrary",),
        ),
    )(W_r1, b_r1, W_r2, b_r2, W_e1, b_e1, W_e2, b_e2, W_e3, b_e3, W_e4,
      b_e4, x, y)

    return out_x, out_y


# speedup vs baseline: 4.2754x; 1.2504x over previous
import jax
import jax.numpy as jnp
from jax.experimental import pallas as pl
from jax.experimental.pallas import tpu as pltpu


def _body(wr1, br1, wr2, br2, we1, be1, we2, be2, we3, be3, we4, be4,
          xr, yr, ox, oy):
    x = [xr[0, c] for c in range(xr.shape[1])]
    y = [yr[0, c] for c in range(yr.shape[1])]
    inv = 1.0 / (x[0].shape[0] * x[0].shape[1])

    mx = [jnp.sum(v) * inv for v in x]
    my = [jnp.sum(v) * inv for v in y]

    def _gates(m):
        h0 = wr1[0, 0] * m[0] + wr1[0, 1] * m[1] + wr1[0, 2] * m[2] \
            + wr1[0, 3] * m[3] + br1[0]
        h1 = wr1[1, 0] * m[0] + wr1[1, 1] * m[1] + wr1[1, 2] * m[2] \
            + wr1[1, 3] * m[3] + br1[1]
        g0 = wr2[0, 0] * h0 + wr2[0, 1] * h1 + br2[0]
        g1 = wr2[1, 0] * h0 + wr2[1, 1] * h1 + br2[1]
        return g0, g1

    gx0, gx1 = _gates(mx)
    gy0, gy1 = _gates(my)
    one = jnp.float32(1.0)
    zero = jnp.float32(0.0)
    mx0 = jnp.where(gx0 > 0, one, zero)
    mx1 = jnp.where(gx1 > 0, one, zero)
    my0 = jnp.where(gy0 > 0, one, zero)
    my1 = jnp.where(gy1 > 0, one, zero)

    co_x, co_y = ox.shape[1], oy.shape[1]
    cx0 = [[mx0 * we1[o, c] for c in range(len(x))] for o in range(co_x)]
    cy0 = [[my0 * we3[o, c] for c in range(len(y))] for o in range(co_x)]
    b0 = [mx0 * be1[o] + my0 * be3[o] for o in range(co_x)]
    cx1 = [[mx1 * we2[o, c] for c in range(len(x))] for o in range(co_y)]
    cy1 = [[my1 * we4[o, c] for c in range(len(y))] for o in range(co_y)]
    b1 = [mx1 * be2[o] + my1 * be4[o] for o in range(co_y)]

    H = x[0].shape[0]
    CH = 16
    for k in range(0, H, CH):
        xc = [v[k:k + CH, :] for v in x]
        yc = [v[k:k + CH, :] for v in y]

        def _mix(cxs, cys, bias):
            acc = cxs[0] * xc[0]
            for c in range(1, len(xc)):
                acc = acc + cxs[c] * xc[c]
            for c in range(len(yc)):
                acc = acc + cys[c] * yc[c]
            return acc + bias

        for o in range(co_x):
            ox[0, o, k:k + CH, :] = _mix(cx0[o], cy0[o], b0[o])
        for o in range(co_y):
            oy[0, o, k:k + CH, :] = _mix(cx1[o], cy1[o], b1[o])


def kernel(x, y, W_r1, b_r1, W_r2, b_r2, W_e1, b_e1, W_e2, b_e2,
           W_e3, b_e3, W_e4, b_e4):
    B, C, H, W = x.shape
    co_x = W_e1.shape[0]
    co_y = W_e2.shape[0]

    smem = pl.BlockSpec(memory_space=pltpu.SMEM)

    def big(c):
        return pl.BlockSpec((1, c, H, W), lambda b: (b, 0, 0, 0))

    out_x, out_y = pl.pallas_call(
        _body,
        grid=(B,),
        in_specs=[smem] * 12 + [big(C), big(C)],
        out_specs=[big(co_x), big(co_y)],
        out_shape=[
            jax.ShapeDtypeStruct((B, co_x, H, W), jnp.float32),
            jax.ShapeDtypeStruct((B, co_y, H, W), jnp.float32),
        ],
        compiler_params=pltpu.CompilerParams(
            dimension_semantics=("arbitrary",),
        ),
    )(W_r1, b_r1, W_r2, b_r2, W_e1, b_e1, W_e2, b_e2, W_e3, b_e3, W_e4,
      b_e4, x, y)

    return out_x, out_y
